# baseline (speedup 1.0000x reference)
import jax
import jax.numpy as jnp
from jax.experimental import pallas as pl
from jax.experimental.pallas import tpu as pltpu

DIMS = 2048
HEAD = 32
MEMORY_SIZE = 64
THRESHOLD = 0.8
CTX = 4096
CHUNK = 128
NCHUNK = CTX // CHUNK


def _body(x_ref, wp_ref, bp_ref, pat_ref, bias_ref, out_ref, acc_ref):
    k = pl.program_id(0)

    @pl.when(k == 0)
    def _init():
        acc_ref[...] = jnp.zeros_like(acc_ref)

    xb = x_ref[...].reshape(x_ref.shape[0], CHUNK // 8, 8, DIMS)
    acc_ref[...] += jnp.sum(xb, axis=1)

    @pl.when(k == NCHUNK - 1)
    def _final():
        pooled = jnp.sum(acc_ref[...], axis=1) * (1.0 / CTX)
        inp = (
            jnp.dot(pooled, wp_ref[...], preferred_element_type=jnp.float32)
            + bp_ref[...][None, :]
        )
        inorm = jnp.sqrt(jnp.sum(inp * inp, axis=1, keepdims=True))
        pat = pat_ref[...]
        pnorm = jnp.sqrt(jnp.sum(pat * pat, axis=1, keepdims=True))
        dots = jax.lax.dot_general(
            inp, pat, (((1,), (1,)), ((), ())),
            preferred_element_type=jnp.float32,
        )
        sims = dots / ((inorm + 1e-8) * (pnorm.T + 1e-8))
        score = jnp.max(sims, axis=1, keepdims=True)
        ids = jax.lax.broadcasted_iota(jnp.int32, sims.shape, 1)
        best = jnp.min(
            jnp.where(sims == score, ids, MEMORY_SIZE), axis=1, keepdims=True
        )
        onehot = (ids == best).astype(jnp.float32)
        sel = jnp.dot(onehot, bias_ref[...], preferred_element_type=jnp.float32)
        gate = jax.nn.sigmoid(score) > THRESHOLD
        out_ref[...] = jnp.where(gate, sel, jnp.zeros_like(sel))


@jax.jit
def kernel(x, xa, W_p, b_p, patterns, biases):
    del xa
    B = x.shape[0]
    out = pl.pallas_call(
        _body,
        grid=(NCHUNK,),
        in_specs=[
            pl.BlockSpec((B, CHUNK, DIMS), lambda k: (0, k, 0)),
            pl.BlockSpec((DIMS, DIMS), lambda k: (0, 0)),
            pl.BlockSpec((DIMS,), lambda k: (0,)),
            pl.BlockSpec((MEMORY_SIZE, DIMS), lambda k: (0, 0)),
            pl.BlockSpec((MEMORY_SIZE, HEAD), lambda k: (0, 0)),
        ],
        out_specs=pl.BlockSpec((B, HEAD), lambda k: (0, 0)),
        out_shape=jax.ShapeDtypeStruct((B, HEAD), jnp.float32),
        scratch_shapes=[pltpu.VMEM((B, 8, DIMS), jnp.float32)],
    )(x, W_p, b_p, patterns, biases)
    return out


# speedup vs baseline: 1.0385x; 1.0385x over previous
import jax
import jax.numpy as jnp
from jax.experimental import pallas as pl
from jax.experimental.pallas import tpu as pltpu

DIMS = 2048
HEAD = 32
MEMORY_SIZE = 64
THRESHOLD = 0.8
CTX = 4096
CHUNK = 256
NCHUNK = CTX // CHUNK


def _body(x_ref, wp_ref, bp_ref, pat_ref, bias_ref, out_ref, acc_ref):
    k = pl.program_id(0)

    @pl.when(k == 0)
    def _init():
        acc_ref[...] = jnp.zeros_like(acc_ref)

    xb = x_ref[...].reshape(x_ref.shape[0], CHUNK // 8, 8, DIMS)
    acc_ref[...] += jnp.sum(xb, axis=1)

    @pl.when(k == NCHUNK - 1)
    def _final():
        pooled = jnp.sum(acc_ref[...], axis=1) * (1.0 / CTX)
        inp = (
            jnp.dot(pooled, wp_ref[...], preferred_element_type=jnp.float32)
            + bp_ref[...][None, :]
        )
        inorm = jnp.sqrt(jnp.sum(inp * inp, axis=1, keepdims=True))
        pat = pat_ref[...]
        pnorm = jnp.sqrt(jnp.sum(pat * pat, axis=1, keepdims=True))
        dots = jax.lax.dot_general(
            inp, pat, (((1,), (1,)), ((), ())),
            preferred_element_type=jnp.float32,
        )
        sims = dots / ((inorm + 1e-8) * (pnorm.T + 1e-8))
        score = jnp.max(sims, axis=1, keepdims=True)
        ids = jax.lax.broadcasted_iota(jnp.int32, sims.shape, 1)
        best = jnp.min(
            jnp.where(sims == score, ids, MEMORY_SIZE), axis=1, keepdims=True
        )
        onehot = (ids == best).astype(jnp.float32)
        sel = jnp.dot(onehot, bias_ref[...], preferred_element_type=jnp.float32)
        gate = jax.nn.sigmoid(score) > THRESHOLD
        out_ref[...] = jnp.where(gate, sel, jnp.zeros_like(sel))


@jax.jit
def kernel(x, xa, W_p, b_p, patterns, biases):
    del xa
    B = x.shape[0]
    out = pl.pallas_call(
        _body,
        grid=(NCHUNK,),
        in_specs=[
            pl.BlockSpec((B, CHUNK, DIMS), lambda k: (0, k, 0)),
            pl.BlockSpec((DIMS, DIMS), lambda k: (0, 0)),
            pl.BlockSpec((DIMS,), lambda k: (0,)),
            pl.BlockSpec((MEMORY_SIZE, DIMS), lambda k: (0, 0)),
            pl.BlockSpec((MEMORY_SIZE, HEAD), lambda k: (0, 0)),
        ],
        out_specs=pl.BlockSpec((B, HEAD), lambda k: (0, 0)),
        out_shape=jax.ShapeDtypeStruct((B, HEAD), jnp.float32),
        scratch_shapes=[pltpu.VMEM((B, 8, DIMS), jnp.float32)],
    )(x, W_p, b_p, patterns, biases)
    return out
